# baseline (speedup 1.0000x reference)
import functools

import jax
import jax.numpy as jnp
from jax import lax
from jax.experimental import pallas as pl
from jax.experimental.pallas import tpu as pltpu
from jax.experimental.pallas import tpu_sc as plsc

D = 300
DP = 384
C = 64


@functools.lru_cache(maxsize=None)
def _make_kernel(B):
    info = plsc.get_sparse_core_info()
    NC, NS = info.num_cores, info.num_subcores
    NW = NC * NS
    assert B % (NW * C) == 0
    b_per_w = B // NW
    n_chunks = b_per_w // C

    mesh = plsc.VectorSubcoreMesh(core_axis_name="c", subcore_axis_name="s")

    @functools.partial(
        pl.kernel,
        mesh=mesh,
        out_type=jax.ShapeDtypeStruct((B, DP), jnp.float32),
        scratch_types=[
            pltpu.VMEM((2, C), jnp.int32),
            pltpu.VMEM((2, C, DP), jnp.float32),
            pltpu.SemaphoreType.DMA,
            pltpu.SemaphoreType.DMA,
            pltpu.SemaphoreType.DMA,
            pltpu.SemaphoreType.DMA,
            pltpu.SemaphoreType.DMA,
        ],
        compiler_params=pltpu.CompilerParams(use_tc_tiling_on_sc=True),
    )
    def k(idx_hbm, table_hbm, out_hbm, idx_v, rows_v, isem, g0, g1, o0, o1):
        wid = lax.axis_index("s") * NC + lax.axis_index("c")
        base = wid * b_per_w
        gsem = (g0, g1)
        osem = (o0, o1)

        def wait_idx(b):
            pltpu.make_async_copy(
                idx_hbm.at[pl.ds(base, C)], idx_v.at[b], isem
            ).wait()

        def drain_store(b):
            pltpu.make_async_copy(
                rows_v.at[b], out_hbm.at[pl.ds(base, C)], osem[b]
            ).wait()

        def wait_gather(b):
            pltpu.make_async_copy(
                table_hbm.at[idx_v.at[b]], rows_v.at[b], gsem[b]
            ).wait()

        pltpu.async_copy(idx_hbm.at[pl.ds(base, C)], idx_v.at[0], isem)
        wait_idx(0)
        pltpu.async_copy(table_hbm.at[idx_v.at[0]], rows_v.at[0], gsem[0])
        pltpu.async_copy(idx_hbm.at[pl.ds(base + C, C)], idx_v.at[1], isem)

        def body(i, carry):
            for b in range(2):
                c = 2 * i + b
                nb = 1 - b

                @pl.when(c + 1 < n_chunks)
                def _():
                    wait_idx(nb)
                    @pl.when(c >= 1)
                    def _():
                        drain_store(nb)

                    pltpu.async_copy(
                        table_hbm.at[idx_v.at[nb]], rows_v.at[nb], gsem[nb]
                    )

                wait_gather(b)

                @pl.when(c + 2 < n_chunks)
                def _():
                    pltpu.async_copy(
                        idx_hbm.at[pl.ds(base + (c + 2) * C, C)],
                        idx_v.at[b],
                        isem,
                    )

                pltpu.async_copy(
                    rows_v.at[b], out_hbm.at[pl.ds(base + c * C, C)], osem[b]
                )
            return carry

        lax.fori_loop(0, n_chunks // 2, body, 0)
        drain_store(0)
        drain_store(1)

    return k


def kernel(x, table):
    bsz, seq = x.shape
    B = bsz * seq
    idx = x.reshape(B).astype(jnp.int32)
    table_p = jnp.pad(table, ((0, 0), (0, DP - D)))
    out = _make_kernel(B)(idx, table_p)
    return out[:, :D].reshape(bsz, seq, D)


# speedup vs baseline: 1.0014x; 1.0014x over previous
import functools

import jax
import jax.numpy as jnp
from jax import lax
from jax.experimental import pallas as pl
from jax.experimental.pallas import tpu as pltpu
from jax.experimental.pallas import tpu_sc as plsc

D = 300
DP = 384
C = 128


@functools.lru_cache(maxsize=None)
def _make_kernel(B):
    info = plsc.get_sparse_core_info()
    NC, NS = info.num_cores, info.num_subcores
    NW = NC * NS
    assert B % (NW * C) == 0
    b_per_w = B // NW
    n_chunks = b_per_w // C

    mesh = plsc.VectorSubcoreMesh(core_axis_name="c", subcore_axis_name="s")

    @functools.partial(
        pl.kernel,
        mesh=mesh,
        out_type=jax.ShapeDtypeStruct((B, DP), jnp.float32),
        scratch_types=[
            pltpu.VMEM((2, C), jnp.int32),
            pltpu.VMEM((2, C, DP), jnp.float32),
            pltpu.SemaphoreType.DMA,
            pltpu.SemaphoreType.DMA,
            pltpu.SemaphoreType.DMA,
            pltpu.SemaphoreType.DMA,
            pltpu.SemaphoreType.DMA,
        ],
        compiler_params=pltpu.CompilerParams(use_tc_tiling_on_sc=True),
    )
    def k(idx_hbm, table_hbm, out_hbm, idx_v, rows_v, isem, g0, g1, o0, o1):
        wid = lax.axis_index("s") * NC + lax.axis_index("c")
        base = wid * b_per_w
        gsem = (g0, g1)
        osem = (o0, o1)

        def wait_idx(b):
            pltpu.make_async_copy(
                idx_hbm.at[pl.ds(base, C)], idx_v.at[b], isem
            ).wait()

        def drain_store(b):
            pltpu.make_async_copy(
                rows_v.at[b], out_hbm.at[pl.ds(base, C)], osem[b]
            ).wait()

        def wait_gather(b):
            pltpu.make_async_copy(
                table_hbm.at[idx_v.at[b]], rows_v.at[b], gsem[b]
            ).wait()

        pltpu.async_copy(idx_hbm.at[pl.ds(base, C)], idx_v.at[0], isem)
        wait_idx(0)
        pltpu.async_copy(table_hbm.at[idx_v.at[0]], rows_v.at[0], gsem[0])
        pltpu.async_copy(idx_hbm.at[pl.ds(base + C, C)], idx_v.at[1], isem)

        def body(i, carry):
            for b in range(2):
                c = 2 * i + b
                nb = 1 - b

                @pl.when(c + 1 < n_chunks)
                def _():
                    wait_idx(nb)
                    @pl.when(c >= 1)
                    def _():
                        drain_store(nb)

                    pltpu.async_copy(
                        table_hbm.at[idx_v.at[nb]], rows_v.at[nb], gsem[nb]
                    )

                wait_gather(b)

                @pl.when(c + 2 < n_chunks)
                def _():
                    pltpu.async_copy(
                        idx_hbm.at[pl.ds(base + (c + 2) * C, C)],
                        idx_v.at[b],
                        isem,
                    )

                pltpu.async_copy(
                    rows_v.at[b], out_hbm.at[pl.ds(base + c * C, C)], osem[b]
                )
            return carry

        lax.fori_loop(0, n_chunks // 2, body, 0)
        drain_store(0)
        drain_store(1)

    return k


def kernel(x, table):
    bsz, seq = x.shape
    B = bsz * seq
    idx = x.reshape(B).astype(jnp.int32)
    table_p = jnp.pad(table, ((0, 0), (0, DP - D)))
    out = _make_kernel(B)(idx, table_p)
    return out[:, :D].reshape(bsz, seq, D)
